# baseline (speedup 1.0000x reference)
import functools

import jax
import jax.numpy as jnp
from jax import lax
from jax.experimental import pallas as pl
from jax.experimental.pallas import tpu as pltpu
from jax.experimental.pallas import tpu_sc as plsc


_BLK = 1024


def _elu(x):
    return jnp.where(x > 0, x, jnp.exp(x) - 1.0)


def _sigmoid(x):
    return 1.0 / (1.0 + jnp.exp(-x))


def _proj_body(x_ref, w_ref, b_ref, o_ref):
    xb = x_ref[...].astype(jnp.bfloat16)
    wb = w_ref[...].astype(jnp.bfloat16)
    o_ref[...] = _elu(
        jnp.dot(xb, wb, preferred_element_type=jnp.float32) + b_ref[...]
    )


def _tc_proj(x, w, b):
    n, d = x.shape
    h = w.shape[1]
    return pl.pallas_call(
        _proj_body,
        grid=(pl.cdiv(n, _BLK),),
        in_specs=[
            pl.BlockSpec((_BLK, d), lambda i: (i, 0)),
            pl.BlockSpec((d, h), lambda i: (0, 0)),
            pl.BlockSpec((1, h), lambda i: (0, 0)),
        ],
        out_specs=pl.BlockSpec((_BLK, h), lambda i: (i, 0)),
        out_shape=jax.ShapeDtypeStruct((n, h), jnp.float32),
    )(x, w, b.reshape(1, -1))


def _layer_body(sa_ref, s2_ref, s0_ref, w11_ref, w21_ref, w01_ref, wu_ref,
                bu_ref, o_ref):
    dot = functools.partial(jnp.dot, preferred_element_type=jnp.float32)
    bf = jnp.bfloat16
    agg = (
        _elu(dot(sa_ref[...].astype(bf), w11_ref[...].astype(bf)))
        + _elu(dot(s2_ref[...].astype(bf), w21_ref[...].astype(bf)))
        + _elu(dot(s0_ref[...].astype(bf), w01_ref[...].astype(bf)))
    )
    o_ref[...] = _sigmoid(
        dot(agg.astype(bf), wu_ref[...].astype(bf)) + bu_ref[...]
    )


def _tc_layer(sa, s2, s0, w11, w21, w01, wu, bu):
    n, h = sa.shape
    mat = pl.BlockSpec((h, h), lambda i: (0, 0))
    row = pl.BlockSpec((_BLK, h), lambda i: (i, 0))
    return pl.pallas_call(
        _layer_body,
        grid=(pl.cdiv(n, _BLK),),
        in_specs=[row, row, row, mat, mat, mat, mat,
                  pl.BlockSpec((1, h), lambda i: (0, 0))],
        out_specs=row,
        out_shape=jax.ShapeDtypeStruct((n, h), jnp.float32),
    )(sa, s2, s0, w11, w21, w01, wu, bu.reshape(1, -1))


def _lin_body(x_ref, w_ref, b_ref, o_ref):
    o_ref[...] = (
        jnp.dot(x_ref[...].astype(jnp.bfloat16), w_ref[...].astype(jnp.bfloat16),
                preferred_element_type=jnp.float32)
        + b_ref[...]
    )


def _tc_lin(x, w, b):
    n, h = x.shape
    c = w.shape[1]
    return pl.pallas_call(
        _lin_body,
        grid=(pl.cdiv(n, _BLK),),
        in_specs=[
            pl.BlockSpec((_BLK, h), lambda i: (i, 0)),
            pl.BlockSpec((h, c), lambda i: (0, 0)),
            pl.BlockSpec((1, c), lambda i: (0, 0)),
        ],
        out_specs=pl.BlockSpec((_BLK, c), lambda i: (i, 0)),
        out_shape=jax.ShapeDtypeStruct((n, c), jnp.float32),
    )(x, w, b.reshape(1, -1))


_NC, _NS = 2, 16
_E = 256
_EB = _E // 128
_RANGE = 6272
_TRASH = _RANGE
_ZR = 64


def _sc_spmm(rows1d, cols1d, bnd, x, n_out):
    n_ranges = -(-n_out // _RANGE)
    assert n_ranges + 1 <= 32 and _RANGE % 128 == 0 and n_out % 16 == 0
    dt = x.dtype
    mesh = plsc.VectorSubcoreMesh(core_axis_name="c", subcore_axis_name="s")

    @functools.partial(
        pl.kernel,
        out_type=jax.ShapeDtypeStruct((n_out, 128), dt),
        mesh=mesh,
        scratch_types=[
            pltpu.VMEM_SHARED((_RANGE + 8, 128), dt),
            pltpu.VMEM((2, _E, 128), dt),
            pltpu.VMEM((2, _E), jnp.int32),
            pltpu.VMEM((2, _E), jnp.int32),
            pltpu.VMEM((2 * _EB, 128), jnp.int32),
            pltpu.VMEM((32,), jnp.int32),
            pltpu.VMEM((_ZR, 128), dt),
            pltpu.SemaphoreType.DMA,
            pltpu.SemaphoreType.DMA,
            pltpu.SemaphoreType.DMA,
        ],
    )
    def spmm(rows_h, cols_h, bnd_h, x_h, z_h, out_h,
             acc, g_v, ridx, cidx, lidx, bnd_v, zero_v, sem, sem_i, sem_s):
        c = lax.axis_index("c")
        s = lax.axis_index("s")
        pltpu.sync_copy(bnd_h, bnd_v)
        pltpu.sync_copy(z_h, zero_v)

        b0 = bnd_v[pl.ds(0, 16)]
        b1 = bnd_v[pl.ds(16, 16)]

        def bval(i):
            return b0[i] if i < 16 else b1[i - 16]

        for rb in range(n_ranges):
            lo = rb * _RANGE
            nrows = min(_RANGE, n_out - lo)

            @pl.when(c == (rb % _NC))
            def _range(rb=rb, lo=lo, nrows=nrows):
                plsc.subcore_barrier()
                share = _RANGE // _NS
                zbase = s * share
                for t in range(-(-share // _ZR)):
                    cnt = min(_ZR, share - t * _ZR)
                    pltpu.sync_copy(zero_v.at[pl.ds(0, cnt)],
                                    acc.at[pl.ds(zbase + t * _ZR, cnt)])
                plsc.subcore_barrier()

                b_lo = bval(rb)
                b_hi = bval(rb + 1)
                a128 = b_lo >> 7
                nsub = (b_hi + 127 - (a128 << 7)) >> 7
                nchunks = (nsub + _EB - 1) // _EB
                nj = (nchunks - s + _NS - 1) // _NS

                def fire_idx(j, b):
                    e0 = (a128 + (s + j * _NS) * _EB) * 128
                    pltpu.async_copy(rows_h.at[pl.ds(e0, _E)],
                                     ridx.at[b], sem_i)
                    pltpu.async_copy(cols_h.at[pl.ds(e0, _E)],
                                     cidx.at[b], sem_i)

                def wait_idx(b):
                    pltpu.make_async_copy(rows_h.at[pl.ds(0, _E)],
                                          ridx.at[b], sem_i).wait()
                    pltpu.make_async_copy(cols_h.at[pl.ds(0, _E)],
                                          cidx.at[b], sem_i).wait()

                def drain_scatter(b):
                    for u in range(_EB):
                        pltpu.make_async_copy(
                            x_h.at[pl.ds(0, 128)],
                            g_v.at[b, pl.ds(u * 128, 128)], sem_s).wait()

                @pl.when(nj > 0)
                def _prime():
                    fire_idx(0, 0)

                def chunk(j, carry):
                    b = j & 1
                    wait_idx(b)

                    @pl.when(j + 1 < nj)
                    def _next():
                        fire_idx(j + 1, 1 - b)

                    gds = [
                        pltpu.async_copy(
                            x_h.at[cidx.at[b, pl.ds(u * 128, 128)]],
                            g_v.at[b, pl.ds(u * 128, 128)], sem)
                        for u in range(_EB)
                    ]

                    def lrc(v, carry2):
                        rv = ridx[b, pl.ds(v * 16, 16)]
                        valid = (rv >= lo) & (rv < lo + nrows)
                        lidx[b * _EB + (v >> 3), pl.ds((v & 7) * 16, 16)] = (
                            jnp.where(valid, rv - lo, _TRASH))
                        return carry2

                    lax.fori_loop(0, _E // 16, lrc, 0)
                    for d in gds:
                        d.wait()

                    @pl.when(j > 0)
                    def _drain_prev():
                        drain_scatter(1 - b)

                    for u in range(_EB):
                        pltpu.async_copy(g_v.at[b, pl.ds(u * 128, 128)],
                                         acc.at[lidx.at[b * _EB + u]],
                                         sem_s, add=True)
                    return carry

                lax.fori_loop(0, nj, chunk, 0)

                @pl.when(nj > 0)
                def _drain_last():
                    drain_scatter((nj - 1) & 1)

                plsc.subcore_barrier()

                wshare = (-(-nrows // _NS) + 15) & ~15
                for t in range(_NS):
                    wb = t * wshare
                    cnt = min(wshare, nrows - wb)
                    if cnt > 0:
                        @pl.when(s == t)
                        def _wb(wb=wb, cnt=cnt):
                            pltpu.sync_copy(
                                acc.at[pl.ds(wb, cnt)],
                                out_h.at[pl.ds(lo + wb, cnt)])

    return spmm(rows1d, cols1d, bnd, x, jnp.zeros((_ZR, 128), dt))


def _prep_idx(rows, cols, n_out):
    nnz = rows.shape[0]
    n_ranges = -(-n_out // _RANGE)
    starts = jnp.arange(0, (n_ranges + 1) * _RANGE, _RANGE, dtype=jnp.int32)
    bnd = jnp.searchsorted(rows, starts, side="left").astype(jnp.int32)
    bnd = jnp.concatenate(
        [bnd, jnp.full((32 - bnd.shape[0],), nnz, jnp.int32)])
    nnzp = ((nnz + _E) + 127) // 128 * 128
    pad = nnzp - nnz
    rows_p = jnp.concatenate([rows, jnp.full((pad,), n_out, jnp.int32)])
    cols_p = jnp.concatenate([cols, jnp.zeros((pad,), jnp.int32)])
    return rows_p, cols_p, bnd


def kernel(x_0, x_1, x_2, adj1_rows, adj1_cols, inc2_rows, inc2_cols,
           inc1t_rows, inc1t_cols, proj0_w, proj0_b, proj1_w, proj1_b,
           proj2_w, proj2_b, conv11_w, conv21_w, conv01_w, upd_w, upd_b,
           lin0_w, lin0_b, lin1_w, lin1_b, lin2_w, lin2_b):
    n1 = x_1.shape[0]
    h0 = _tc_proj(x_0, proj0_w, proj0_b)
    h1 = _tc_proj(x_1, proj1_w, proj1_b)
    h2 = _tc_proj(x_2, proj2_w, proj2_b)

    a_r, a_c, a_b = _prep_idx(adj1_rows, adj1_cols, n1)
    i2_r, i2_c, i2_b = _prep_idx(inc2_rows, inc2_cols, n1)
    i1_r, i1_c, i1_b = _prep_idx(inc1t_rows, inc1t_cols, n1)

    s0 = _sc_spmm(i1_r, i1_c, i1_b, h0, n1)
    s2 = _sc_spmm(i2_r, i2_c, i2_b, h2, n1)

    for i in range(conv11_w.shape[0]):
        sa = _sc_spmm(a_r, a_c, a_b, h1, n1)
        h1 = _tc_layer(sa, s2, s0, conv11_w[i], conv21_w[i], conv01_w[i],
                       upd_w[i], upd_b[i])

    out0 = _tc_lin(h0, lin0_w, lin0_b)
    out1 = _tc_lin(h1, lin1_w, lin1_b)
    out2 = _tc_lin(h2, lin2_w, lin2_b)
    return (out0, out1, out2)


# speedup vs baseline: 1.0656x; 1.0656x over previous
import functools

import jax
import jax.numpy as jnp
from jax import lax
from jax.experimental import pallas as pl
from jax.experimental.pallas import tpu as pltpu
from jax.experimental.pallas import tpu_sc as plsc


_BLK = 1024


def _elu(x):
    return jnp.where(x > 0, x, jnp.exp(x) - 1.0)


def _sigmoid(x):
    return 1.0 / (1.0 + jnp.exp(-x))


def _dot(a, b):
    return jnp.dot(a, b, preferred_element_type=jnp.float32)


def _proj_head_body(x_ref, w_ref, b_ref, lw_ref, lb_ref, h_ref, o_ref):
    h = _elu(_dot(x_ref[...], w_ref[...]) + b_ref[...])
    h_ref[...] = h
    o_ref[...] = _dot(h, lw_ref[...]) + lb_ref[...]


def _tc_proj_head(x, w, b, lw, lb):
    n, d = x.shape
    h = w.shape[1]
    c = lw.shape[1]
    return pl.pallas_call(
        _proj_head_body,
        grid=(pl.cdiv(n, _BLK),),
        in_specs=[
            pl.BlockSpec((_BLK, d), lambda i: (i, 0)),
            pl.BlockSpec((d, h), lambda i: (0, 0)),
            pl.BlockSpec((1, h), lambda i: (0, 0)),
            pl.BlockSpec((h, c), lambda i: (0, 0)),
            pl.BlockSpec((1, c), lambda i: (0, 0)),
        ],
        out_specs=[
            pl.BlockSpec((_BLK, h), lambda i: (i, 0)),
            pl.BlockSpec((_BLK, c), lambda i: (i, 0)),
        ],
        out_shape=[
            jax.ShapeDtypeStruct((n, h), jnp.float32),
            jax.ShapeDtypeStruct((n, c), jnp.float32),
        ],
    )(x, w, b.reshape(1, -1), lw, lb.reshape(1, -1))


def _proj_body(x_ref, w_ref, b_ref, h_ref):
    h_ref[...] = _elu(_dot(x_ref[...], w_ref[...]) + b_ref[...])


def _tc_proj(x, w, b):
    n, d = x.shape
    h = w.shape[1]
    return pl.pallas_call(
        _proj_body,
        grid=(pl.cdiv(n, _BLK),),
        in_specs=[
            pl.BlockSpec((_BLK, d), lambda i: (i, 0)),
            pl.BlockSpec((d, h), lambda i: (0, 0)),
            pl.BlockSpec((1, h), lambda i: (0, 0)),
        ],
        out_specs=pl.BlockSpec((_BLK, h), lambda i: (i, 0)),
        out_shape=jax.ShapeDtypeStruct((n, h), jnp.float32),
    )(x, w, b.reshape(1, -1))


def _agg(sa_ref, s2_ref, s0_ref, w11_ref, w21_ref, w01_ref, wu_ref, bu_ref):
    agg = (
        _elu(_dot(sa_ref[...], w11_ref[...]))
        + _elu(_dot(s2_ref[...], w21_ref[...]))
        + _elu(_dot(s0_ref[...], w01_ref[...]))
    )
    return _sigmoid(_dot(agg, wu_ref[...]) + bu_ref[...])


def _layer_body(sa_ref, s2_ref, s0_ref, w11_ref, w21_ref, w01_ref, wu_ref,
                bu_ref, o_ref):
    o_ref[...] = _agg(sa_ref, s2_ref, s0_ref, w11_ref, w21_ref, w01_ref,
                      wu_ref, bu_ref)


def _layer_head_body(sa_ref, s2_ref, s0_ref, w11_ref, w21_ref, w01_ref,
                     wu_ref, bu_ref, lw_ref, lb_ref, o_ref):
    h1 = _agg(sa_ref, s2_ref, s0_ref, w11_ref, w21_ref, w01_ref,
              wu_ref, bu_ref)
    o_ref[...] = _dot(h1, lw_ref[...]) + lb_ref[...]


def _tc_layer(sa, s2, s0, w11, w21, w01, wu, bu, head=None):
    n, h = sa.shape
    mat = pl.BlockSpec((h, h), lambda i: (0, 0))
    row = pl.BlockSpec((_BLK, h), lambda i: (i, 0))
    vec = pl.BlockSpec((1, h), lambda i: (0, 0))
    if head is None:
        return pl.pallas_call(
            _layer_body,
            grid=(pl.cdiv(n, _BLK),),
            in_specs=[row, row, row, mat, mat, mat, mat, vec],
            out_specs=row,
            out_shape=jax.ShapeDtypeStruct((n, h), jnp.float32),
        )(sa, s2, s0, w11, w21, w01, wu, bu.reshape(1, -1))
    lw, lb = head
    c = lw.shape[1]
    return pl.pallas_call(
        _layer_head_body,
        grid=(pl.cdiv(n, _BLK),),
        in_specs=[row, row, row, mat, mat, mat, mat, vec,
                  pl.BlockSpec((h, c), lambda i: (0, 0)),
                  pl.BlockSpec((1, c), lambda i: (0, 0))],
        out_specs=pl.BlockSpec((_BLK, c), lambda i: (i, 0)),
        out_shape=jax.ShapeDtypeStruct((n, c), jnp.float32),
    )(sa, s2, s0, w11, w21, w01, wu, bu.reshape(1, -1),
      lw, lb.reshape(1, -1))


_NC, _NS = 2, 16
_E = 256
_EB = _E // 128
_RANGE = 6272
_TRASH = _RANGE
_ZR = 64


def _sc_spmm(rows1d, cols1d, bnd, x, n_out):
    nnz = rows1d.shape[0]
    n_ranges = -(-n_out // _RANGE)
    assert n_ranges + 1 <= 32 and _RANGE % 128 == 0 and n_out % 16 == 0
    assert nnz % 128 == 0 and nnz >= _E
    dt = x.dtype
    mesh = plsc.VectorSubcoreMesh(core_axis_name="c", subcore_axis_name="s")

    @functools.partial(
        pl.kernel,
        out_type=jax.ShapeDtypeStruct((n_out, 128), dt),
        mesh=mesh,
        scratch_types=[
            pltpu.VMEM_SHARED((_RANGE + 8, 128), dt),
            pltpu.VMEM((2, _E, 128), dt),
            pltpu.VMEM((2, _E), jnp.int32),
            pltpu.VMEM((2, _E), jnp.int32),
            pltpu.VMEM((2 * _EB, 128), jnp.int32),
            pltpu.VMEM((32,), jnp.int32),
            pltpu.VMEM((_ZR, 128), dt),
            pltpu.SemaphoreType.DMA,
            pltpu.SemaphoreType.DMA,
            pltpu.SemaphoreType.DMA,
        ],
    )
    def spmm(rows_h, cols_h, bnd_h, x_h, z_h, out_h,
             acc, g_v, ridx, cidx, lidx, bnd_v, zero_v, sem, sem_i, sem_s):
        c = lax.axis_index("c")
        s = lax.axis_index("s")
        pltpu.sync_copy(bnd_h, bnd_v)
        pltpu.sync_copy(z_h, zero_v)

        b0 = bnd_v[pl.ds(0, 16)]
        b1 = bnd_v[pl.ds(16, 16)]

        def bval(i):
            return b0[i] if i < 16 else b1[i - 16]

        for rb in range(n_ranges):
            lo = rb * _RANGE
            nrows = min(_RANGE, n_out - lo)

            @pl.when(c == (rb % _NC))
            def _range(rb=rb, lo=lo, nrows=nrows):
                plsc.subcore_barrier()
                share = _RANGE // _NS
                zbase = s * share
                for t in range(-(-share // _ZR)):
                    cnt = min(_ZR, share - t * _ZR)
                    pltpu.sync_copy(zero_v.at[pl.ds(0, cnt)],
                                    acc.at[pl.ds(zbase + t * _ZR, cnt)])
                plsc.subcore_barrier()

                b_lo = bval(rb)
                b_hi = bval(rb + 1)
                a128 = b_lo >> 7
                nsub = (b_hi + 127 - (a128 << 7)) >> 7
                nchunks = (nsub + _EB - 1) // _EB
                nj = (nchunks - s + _NS - 1) // _NS

                def espan(j):
                    e0 = (a128 + (s + j * _NS) * _EB) * 128
                    e0c = pl.multiple_of(jnp.minimum(e0, nnz - _E), 128)
                    return e0c, e0 - e0c

                def fire_idx(j, b):
                    e0c, _ = espan(j)
                    pltpu.async_copy(rows_h.at[pl.ds(e0c, _E)],
                                     ridx.at[b], sem_i)
                    pltpu.async_copy(cols_h.at[pl.ds(e0c, _E)],
                                     cidx.at[b], sem_i)

                def wait_idx(b):
                    pltpu.make_async_copy(rows_h.at[pl.ds(0, _E)],
                                          ridx.at[b], sem_i).wait()
                    pltpu.make_async_copy(cols_h.at[pl.ds(0, _E)],
                                          cidx.at[b], sem_i).wait()

                def drain_scatter(b):
                    for u in range(_EB):
                        pltpu.make_async_copy(
                            x_h.at[pl.ds(0, 128)],
                            g_v.at[b, pl.ds(u * 128, 128)], sem_s).wait()

                @pl.when(nj > 0)
                def _prime():
                    fire_idx(0, 0)

                def chunk(j, carry):
                    b = j & 1
                    _, shift = espan(j)
                    wait_idx(b)

                    @pl.when(j + 1 < nj)
                    def _next():
                        fire_idx(j + 1, 1 - b)

                    gds = [
                        pltpu.async_copy(
                            x_h.at[cidx.at[b, pl.ds(u * 128, 128)]],
                            g_v.at[b, pl.ds(u * 128, 128)], sem)
                        for u in range(_EB)
                    ]

                    def lrc(v, carry2):
                        rv = ridx[b, pl.ds(v * 16, 16)]
                        pv = lax.broadcasted_iota(
                            jnp.int32, (16,), 0) + v * 16
                        valid = ((rv >= lo) & (rv < lo + nrows)
                                 & (pv >= shift))
                        lidx[b * _EB + (v >> 3), pl.ds((v & 7) * 16, 16)] = (
                            jnp.where(valid, rv - lo, _TRASH))
                        return carry2

                    lax.fori_loop(0, _E // 16, lrc, 0)
                    for d in gds:
                        d.wait()

                    @pl.when(j > 0)
                    def _drain_prev():
                        drain_scatter(1 - b)

                    for u in range(_EB):
                        pltpu.async_copy(g_v.at[b, pl.ds(u * 128, 128)],
                                         acc.at[lidx.at[b * _EB + u]],
                                         sem_s, add=True)
                    return carry

                lax.fori_loop(0, nj, chunk, 0)

                @pl.when(nj > 0)
                def _drain_last():
                    drain_scatter((nj - 1) & 1)

                plsc.subcore_barrier()

                wshare = (-(-nrows // _NS) + 15) & ~15
                for t in range(_NS):
                    wb = t * wshare
                    cnt = min(wshare, nrows - wb)
                    if cnt > 0:
                        @pl.when(s == t)
                        def _wb(wb=wb, cnt=cnt):
                            pltpu.sync_copy(
                                acc.at[pl.ds(wb, cnt)],
                                out_h.at[pl.ds(lo + wb, cnt)])

    return spmm(rows1d, cols1d, bnd, x, jnp.zeros((_ZR, 128), dt))


def _prep_bnd(rows, n_out):
    nnz = rows.shape[0]
    n_ranges = -(-n_out // _RANGE)
    starts = jnp.arange(0, (n_ranges + 1) * _RANGE, _RANGE, dtype=jnp.int32)
    bnd = jnp.searchsorted(rows, starts, side="left").astype(jnp.int32)
    return jnp.concatenate(
        [bnd, jnp.full((32 - bnd.shape[0],), nnz, jnp.int32)])


def kernel(x_0, x_1, x_2, adj1_rows, adj1_cols, inc2_rows, inc2_cols,
           inc1t_rows, inc1t_cols, proj0_w, proj0_b, proj1_w, proj1_b,
           proj2_w, proj2_b, conv11_w, conv21_w, conv01_w, upd_w, upd_b,
           lin0_w, lin0_b, lin1_w, lin1_b, lin2_w, lin2_b):
    n1 = x_1.shape[0]
    h0, out0 = _tc_proj_head(x_0, proj0_w, proj0_b, lin0_w, lin0_b)
    h2, out2 = _tc_proj_head(x_2, proj2_w, proj2_b, lin2_w, lin2_b)
    h1 = _tc_proj(x_1, proj1_w, proj1_b)

    pad = (-inc1t_rows.shape[0]) % 128
    if pad:
        inc1t_rows = jnp.concatenate(
            [inc1t_rows, jnp.full((pad,), n1, jnp.int32)])
        inc1t_cols = jnp.concatenate(
            [inc1t_cols, jnp.zeros((pad,), jnp.int32)])

    a_b = _prep_bnd(adj1_rows, n1)
    i2_b = _prep_bnd(inc2_rows, n1)
    i1_b = _prep_bnd(inc1t_rows, n1)

    s0 = _sc_spmm(inc1t_rows, inc1t_cols, i1_b, h0, n1)
    s2 = _sc_spmm(inc2_rows, inc2_cols, i2_b, h2, n1)

    nl = conv11_w.shape[0]
    out1 = None
    for i in range(nl):
        sa = _sc_spmm(adj1_rows, adj1_cols, a_b, h1, n1)
        head = (lin1_w, lin1_b) if i == nl - 1 else None
        res = _tc_layer(sa, s2, s0, conv11_w[i], conv21_w[i], conv01_w[i],
                        upd_w[i], upd_b[i], head=head)
        if i == nl - 1:
            out1 = res
        else:
            h1 = res

    return (out0, out1, out2)
